# Initial kernel scaffold; baseline (speedup 1.0000x reference)
import jax
import jax.numpy as jnp
from jax.experimental import pallas as pl


def kernel(nodes, edges, globals_, senders, receivers, W_embed, b_embed, W_mlp, b_mlp, ln_scale, ln_offset, W_dec, b_dec):
    raise NotImplementedError("write your pallas kernel here")


# speedup vs baseline: 4.4023x; 4.4023x over previous
import functools

import jax
import jax.numpy as jnp
from jax import lax
from jax.experimental import pallas as pl
from jax.experimental.pallas import tpu as pltpu
from jax.experimental.pallas import tpu_sc as plsc

NC = 2
NS = 16
NW = NC * NS
K = 128

_PREC = lax.Precision.HIGHEST
_SC_PARAMS = pltpu.CompilerParams(use_tc_tiling_on_sc=False)


def _make_scatter(n_acc, epad, d2):
    epw = epad // NW
    nch = epw // K
    rpw = n_acc // NS
    mesh = plsc.VectorSubcoreMesh(core_axis_name="c", subcore_axis_name="s")

    @functools.partial(
        pl.kernel,
        out_type=tuple(jax.ShapeDtypeStruct((n_acc, d2), jnp.float32)
                       for _ in range(4)),
        mesh=mesh,
        compiler_params=_SC_PARAMS,
        scratch_types=[
            pltpu.VMEM((K,), jnp.int32),
            pltpu.VMEM((K,), jnp.int32),
            pltpu.VMEM((K, d2), jnp.float32),
            pltpu.VMEM_SHARED((n_acc, d2), jnp.float32),
            pltpu.VMEM_SHARED((n_acc, d2), jnp.float32),
        ],
    )
    def scatter_k(xs0_hbm, xs1_hbm, snd_hbm, rcv_hbm, zer_hbm,
                  out00, out01, out10, out11,
                  idx_s, idx_r, rows, xs_sh, acc):
        cid = lax.axis_index("c")
        sid = lax.axis_index("s")
        wid = sid * NC + cid
        r0 = sid * rpw

        def one_half(xs_hbm, outA, outB):
            pltpu.sync_copy(xs_hbm.at[pl.ds(r0, rpw)], xs_sh.at[pl.ds(r0, rpw)])
            pltpu.sync_copy(zer_hbm.at[pl.ds(r0, rpw)], acc.at[pl.ds(r0, rpw)])
            plsc.subcore_barrier()

            def body(ch, carry):
                base = wid * epw + ch * K
                pltpu.sync_copy(snd_hbm.at[pl.ds(base, K)], idx_s)
                pltpu.sync_copy(rcv_hbm.at[pl.ds(base, K)], idx_r)
                pltpu.sync_copy(xs_sh.at[idx_s], rows)
                pltpu.sync_copy(rows, acc.at[idx_r], add=True)
                return carry

            lax.fori_loop(0, nch, body, 0)
            plsc.subcore_barrier()

            @pl.when(cid == 0)
            def _():
                pltpu.sync_copy(acc.at[pl.ds(r0, rpw)], outA.at[pl.ds(r0, rpw)])

            @pl.when(cid == 1)
            def _():
                pltpu.sync_copy(acc.at[pl.ds(r0, rpw)], outB.at[pl.ds(r0, rpw)])

            plsc.subcore_barrier()

        one_half(xs0_hbm, out00, out01)
        one_half(xs1_hbm, out10, out11)

    return scatter_k


def _make_degree(drows, m):
    mpw = m // NW
    nch = mpw // K
    rpw = drows // NS
    mesh = plsc.VectorSubcoreMesh(core_axis_name="c", subcore_axis_name="s")

    @functools.partial(
        pl.kernel,
        out_type=(jax.ShapeDtypeStruct((drows, 16), jnp.float32),
                  jax.ShapeDtypeStruct((drows, 16), jnp.float32)),
        mesh=mesh,
        compiler_params=_SC_PARAMS,
        scratch_types=[
            pltpu.VMEM((K,), jnp.int32),
            pltpu.VMEM((K, 16), jnp.float32),
            pltpu.VMEM_SHARED((drows, 16), jnp.float32),
        ],
    )
    def degree_k(didx_hbm, ones_hbm, zer_hbm, out0, out1, idx, ones_v, dacc):
        cid = lax.axis_index("c")
        sid = lax.axis_index("s")
        wid = sid * NC + cid
        r0 = sid * rpw
        pltpu.sync_copy(ones_hbm, ones_v)
        pltpu.sync_copy(zer_hbm.at[pl.ds(r0, rpw)], dacc.at[pl.ds(r0, rpw)])
        plsc.subcore_barrier()

        def body(ch, carry):
            base = wid * mpw + ch * K
            pltpu.sync_copy(didx_hbm.at[pl.ds(base, K)], idx)
            pltpu.sync_copy(ones_v, dacc.at[idx], add=True)
            return carry

        lax.fori_loop(0, nch, body, 0)
        plsc.subcore_barrier()

        @pl.when(cid == 0)
        def _():
            pltpu.sync_copy(dacc.at[pl.ds(r0, rpw)], out0.at[pl.ds(r0, rpw)])

        @pl.when(cid == 1)
        def _():
            pltpu.sync_copy(dacc.at[pl.ds(r0, rpw)], out1.at[pl.ds(r0, rpw)])

    return degree_k


def _mlp_block(x, w0, b0, w1, b1):
    x = jnp.maximum(jnp.dot(x, w0, precision=_PREC) + b0, 0.0)
    return jnp.maximum(jnp.dot(x, w1, precision=_PREC) + b1, 0.0)


def _f0_body(d2, nodes, we, be, w0, b0, w1, b1, ds0, ds1,
             h_out, xs0_out, xs1_out):
    h = jnp.dot(nodes[...], we[...], precision=_PREC) + be[...]
    h_out[...] = h
    x = _mlp_block(h, w0[...], b0[...], w1[...], b1[...])
    xs = x * lax.rsqrt(ds0[...] + ds1[...] + 1.0)
    xs0_out[...] = xs[:, :d2]
    xs1_out[...] = xs[:, d2:]


def _f1_body(d2, a00, a01, a10, a11, xs0, xs1, h, dr0, dr1, lnsc, lnof,
             w0, b0, w1, b1, ds0, ds1, h_out, xs0_out, xs1_out):
    y = jnp.concatenate([a00[...] + a01[...] + xs0[...],
                         a10[...] + a11[...] + xs1[...]], axis=1)
    y = y * lax.rsqrt(dr0[...] + dr1[...] + 1.0)
    t = y + h[...]
    mean = jnp.mean(t, axis=1, keepdims=True)
    var = jnp.mean((t - mean) ** 2, axis=1, keepdims=True)
    hn = (t - mean) * lax.rsqrt(var + 1e-5) * lnsc[...] + lnof[...]
    h_out[...] = hn
    x = _mlp_block(hn, w0[...], b0[...], w1[...], b1[...])
    xs = x * lax.rsqrt(ds0[...] + ds1[...] + 1.0)
    xs0_out[...] = xs[:, :d2]
    xs1_out[...] = xs[:, d2:]


def _f2_body(n_valid, grid, a00, a01, a10, a11, xs0, xs1, h, dr0, dr1,
             lnsc, lnof, e2d, wd, bd, h_out, e_out, g_out, acc_ref):
    i = pl.program_id(0)
    y = jnp.concatenate([a00[...] + a01[...] + xs0[...],
                         a10[...] + a11[...] + xs1[...]], axis=1)
    y = y * lax.rsqrt(dr0[...] + dr1[...] + 1.0)
    t = y + h[...]
    mean = jnp.mean(t, axis=1, keepdims=True)
    var = jnp.mean((t - mean) ** 2, axis=1, keepdims=True)
    hn = (t - mean) * lax.rsqrt(var + 1e-5) * lnsc[...] + lnof[...]
    h_out[...] = hn
    e_out[...] = e2d[...] * 4.0

    blk = hn.shape[0]
    row = i * blk + lax.broadcasted_iota(jnp.int32, (blk, 1), 0)
    masked = jnp.where(row < n_valid, hn, 0.0)
    psum = jnp.sum(masked, axis=0, keepdims=True)

    @pl.when(i == 0)
    def _():
        acc_ref[...] = jnp.zeros_like(acc_ref)

    acc_ref[...] += psum

    @pl.when(i == grid - 1)
    def _():
        pooled = acc_ref[...] * (1.0 / n_valid)
        g_out[...] = jnp.dot(pooled, wd[...], precision=_PREC) + bd[...]


def kernel(nodes, edges, globals_, senders, receivers, W_embed, b_embed,
           W_mlp, b_mlp, ln_scale, ln_offset, W_dec, b_dec):
    n, d = nodes.shape
    e = senders.shape[0]
    latent = W_embed.shape[1]
    out_g = W_dec.shape[1]
    d2 = latent // 2

    n_acc = ((n + 1023) // 1024 + (0 if n % 1024 else 1)) * 1024
    if n_acc <= n:
        n_acc = n + 1024
    ndum = n_acc - n
    epad = ((e + NW * K - 1) // (NW * K)) * (NW * K)
    npad = epad - e
    drows = 2 * n_acc

    pad = (jnp.arange(npad, dtype=jnp.int32) % ndum) + n
    snd = jnp.concatenate([senders.astype(jnp.int32), pad])
    rcv = jnp.concatenate([receivers.astype(jnp.int32), pad])
    didx = jnp.concatenate([snd, rcv + n_acc])

    zer_half = jnp.zeros((n_acc, d2), jnp.float32)
    zer_deg = jnp.zeros((drows, 16), jnp.float32)
    ones16 = jnp.ones((K, 16), jnp.float32)

    deg_k = _make_degree(drows, 2 * epad)
    d0, d1 = deg_k(didx, ones16, zer_deg)
    ds0 = d0[:n_acc, 0:1]
    ds1 = d1[:n_acc, 0:1]
    dr0 = d0[n_acc:, 0:1]
    dr1 = d1[n_acc:, 0:1]

    nodes_pad = jnp.pad(nodes, ((0, n_acc - n), (0, 0)))
    be = b_embed.reshape(1, latent)
    bd = b_dec.reshape(1, out_g)

    grid = 16
    blk = n_acc // grid
    row_spec = pl.BlockSpec((blk, latent), lambda i: (i, 0))
    half_spec = pl.BlockSpec((blk, d2), lambda i: (i, 0))
    col_spec = pl.BlockSpec((blk, 1), lambda i: (i, 0))
    full_spec = pl.BlockSpec((d, latent), lambda i: (0, 0))
    vec_spec = pl.BlockSpec((1, latent), lambda i: (0, 0))

    half_shape = jax.ShapeDtypeStruct((n_acc, d2), jnp.float32)
    full_shape = jax.ShapeDtypeStruct((n_acc, latent), jnp.float32)

    f0 = pl.pallas_call(
        functools.partial(_f0_body, d2),
        grid=(grid,),
        in_specs=[row_spec, full_spec, vec_spec, full_spec, vec_spec,
                  full_spec, vec_spec, col_spec, col_spec],
        out_specs=[row_spec, half_spec, half_spec],
        out_shape=[full_shape, half_shape, half_shape],
    )
    h0, xs00, xs01 = f0(nodes_pad, W_embed, be,
                        W_mlp[0, 0], b_mlp[0, 0].reshape(1, -1),
                        W_mlp[0, 1], b_mlp[0, 1].reshape(1, -1), ds0, ds1)

    scat_k = _make_scatter(n_acc, epad, d2)

    a000, a001, a010, a011 = scat_k(xs00, xs01, snd, rcv, zer_half)
    f1 = pl.pallas_call(
        functools.partial(_f1_body, d2),
        grid=(grid,),
        in_specs=[half_spec, half_spec, half_spec, half_spec, half_spec,
                  half_spec, row_spec, col_spec, col_spec,
                  vec_spec, vec_spec, full_spec, vec_spec, full_spec,
                  vec_spec, col_spec, col_spec],
        out_specs=[row_spec, half_spec, half_spec],
        out_shape=[full_shape, half_shape, half_shape],
    )
    h1, xs10, xs11 = f1(a000, a001, a010, a011, xs00, xs01, h0, dr0, dr1,
                        ln_scale[0].reshape(1, -1), ln_offset[0].reshape(1, -1),
                        W_mlp[1, 0], b_mlp[1, 0].reshape(1, -1),
                        W_mlp[1, 1], b_mlp[1, 1].reshape(1, -1), ds0, ds1)

    a100, a101, a110, a111 = scat_k(xs10, xs11, snd, rcv, zer_half)

    e4r = e * edges.shape[1] // latent
    g2 = 10
    eblk = e4r // g2
    blk2 = n_acc // g2
    row2 = pl.BlockSpec((blk2, latent), lambda i: (i, 0))
    half2 = pl.BlockSpec((blk2, d2), lambda i: (i, 0))
    col2 = pl.BlockSpec((blk2, 1), lambda i: (i, 0))
    vec2 = pl.BlockSpec((1, latent), lambda i: (0, 0))
    e2d = edges.reshape(e4r, latent)

    f2 = pl.pallas_call(
        functools.partial(_f2_body, float(n), g2),
        grid=(g2,),
        in_specs=[half2, half2, half2, half2, half2, half2, row2,
                  col2, col2, vec2, vec2,
                  pl.BlockSpec((eblk, latent), lambda i: (i, 0)),
                  pl.BlockSpec((latent, out_g), lambda i: (0, 0)),
                  pl.BlockSpec((1, out_g), lambda i: (0, 0))],
        out_specs=[row2,
                   pl.BlockSpec((eblk, latent), lambda i: (i, 0)),
                   pl.BlockSpec((1, out_g), lambda i: (0, 0))],
        out_shape=[jax.ShapeDtypeStruct((n_acc, latent), jnp.float32),
                   jax.ShapeDtypeStruct((e4r, latent), jnp.float32),
                   jax.ShapeDtypeStruct((1, out_g), jnp.float32)],
        scratch_shapes=[pltpu.VMEM((1, latent), jnp.float32)],
    )
    h2, eout, g = f2(a100, a101, a110, a111, xs10, xs11, h1, dr0, dr1,
                     ln_scale[1].reshape(1, -1), ln_offset[1].reshape(1, -1),
                     e2d, W_dec, bd)

    return (h2[:n], eout.reshape(e, edges.shape[1]), g)


# speedup vs baseline: 5.9739x; 1.3570x over previous
import functools

import jax
import jax.numpy as jnp
from jax import lax
from jax.experimental import pallas as pl
from jax.experimental.pallas import tpu as pltpu
from jax.experimental.pallas import tpu_sc as plsc

NC = 2
NS = 16
NW = NC * NS
K = 512

_PREC = lax.Precision.HIGHEST
_SC_PARAMS = pltpu.CompilerParams(use_tc_tiling_on_sc=False)


def _make_scatter(n_acc, epad, d2):
    epw = epad // NW
    nch = epw // K
    rpw = n_acc // NS
    mesh = plsc.VectorSubcoreMesh(core_axis_name="c", subcore_axis_name="s")

    @functools.partial(
        pl.kernel,
        out_type=tuple(jax.ShapeDtypeStruct((n_acc, d2), jnp.float32)
                       for _ in range(4)),
        mesh=mesh,
        compiler_params=_SC_PARAMS,
        scratch_types=[
            pltpu.VMEM((K,), jnp.int32),
            pltpu.VMEM((K,), jnp.int32),
            pltpu.VMEM((K, d2), jnp.float32),
            pltpu.VMEM_SHARED((n_acc, d2), jnp.float32),
            pltpu.VMEM_SHARED((n_acc, d2), jnp.float32),
        ],
    )
    def scatter_k(xs0_hbm, xs1_hbm, snd_hbm, rcv_hbm, zer_hbm,
                  out00, out01, out10, out11,
                  idx_s, idx_r, rows, xs_sh, acc):
        cid = lax.axis_index("c")
        sid = lax.axis_index("s")
        wid = sid * NC + cid
        r0 = sid * rpw

        def one_half(xs_hbm, outA, outB):
            pltpu.sync_copy(xs_hbm.at[pl.ds(r0, rpw)], xs_sh.at[pl.ds(r0, rpw)])
            pltpu.sync_copy(zer_hbm.at[pl.ds(r0, rpw)], acc.at[pl.ds(r0, rpw)])
            plsc.subcore_barrier()

            def body(ch, carry):
                base = wid * epw + ch * K
                pltpu.sync_copy(snd_hbm.at[pl.ds(base, K)], idx_s)
                pltpu.sync_copy(rcv_hbm.at[pl.ds(base, K)], idx_r)
                pltpu.sync_copy(xs_sh.at[idx_s], rows)
                pltpu.sync_copy(rows, acc.at[idx_r], add=True)
                return carry

            lax.fori_loop(0, nch, body, 0)
            plsc.subcore_barrier()

            @pl.when(cid == 0)
            def _():
                pltpu.sync_copy(acc.at[pl.ds(r0, rpw)], outA.at[pl.ds(r0, rpw)])

            @pl.when(cid == 1)
            def _():
                pltpu.sync_copy(acc.at[pl.ds(r0, rpw)], outB.at[pl.ds(r0, rpw)])

            plsc.subcore_barrier()

        one_half(xs0_hbm, out00, out01)
        one_half(xs1_hbm, out10, out11)

    return scatter_k


def _make_degree(drows, m):
    mpw = m // NW
    nch = mpw // K
    rpw = drows // NS
    mesh = plsc.VectorSubcoreMesh(core_axis_name="c", subcore_axis_name="s")

    @functools.partial(
        pl.kernel,
        out_type=(jax.ShapeDtypeStruct((drows, 16), jnp.float32),
                  jax.ShapeDtypeStruct((drows, 16), jnp.float32)),
        mesh=mesh,
        compiler_params=_SC_PARAMS,
        scratch_types=[
            pltpu.VMEM((K,), jnp.int32),
            pltpu.VMEM((K, 16), jnp.float32),
            pltpu.VMEM_SHARED((drows, 16), jnp.float32),
        ],
    )
    def degree_k(didx_hbm, ones_hbm, zer_hbm, out0, out1, idx, ones_v, dacc):
        cid = lax.axis_index("c")
        sid = lax.axis_index("s")
        wid = sid * NC + cid
        r0 = sid * rpw
        pltpu.sync_copy(ones_hbm, ones_v)
        pltpu.sync_copy(zer_hbm.at[pl.ds(r0, rpw)], dacc.at[pl.ds(r0, rpw)])
        plsc.subcore_barrier()

        def body(ch, carry):
            base = wid * mpw + ch * K
            pltpu.sync_copy(didx_hbm.at[pl.ds(base, K)], idx)
            pltpu.sync_copy(ones_v, dacc.at[idx], add=True)
            return carry

        lax.fori_loop(0, nch, body, 0)
        plsc.subcore_barrier()

        @pl.when(cid == 0)
        def _():
            pltpu.sync_copy(dacc.at[pl.ds(r0, rpw)], out0.at[pl.ds(r0, rpw)])

        @pl.when(cid == 1)
        def _():
            pltpu.sync_copy(dacc.at[pl.ds(r0, rpw)], out1.at[pl.ds(r0, rpw)])

    return degree_k


def _mlp_block(x, w0, b0, w1, b1):
    x = jnp.maximum(jnp.dot(x, w0, precision=_PREC) + b0, 0.0)
    return jnp.maximum(jnp.dot(x, w1, precision=_PREC) + b1, 0.0)


def _f0_body(d2, nodes, we, be, w0, b0, w1, b1, ds0, ds1,
             h_out, xs0_out, xs1_out):
    h = jnp.dot(nodes[...], we[...], precision=_PREC) + be[...]
    h_out[...] = h
    x = _mlp_block(h, w0[...], b0[...], w1[...], b1[...])
    xs = x * lax.rsqrt(ds0[...] + ds1[...] + 1.0)
    xs0_out[...] = xs[:, :d2]
    xs1_out[...] = xs[:, d2:]


def _f1_body(d2, a00, a01, a10, a11, xs0, xs1, h, dr0, dr1, lnsc, lnof,
             w0, b0, w1, b1, ds0, ds1, h_out, xs0_out, xs1_out):
    y = jnp.concatenate([a00[...] + a01[...] + xs0[...],
                         a10[...] + a11[...] + xs1[...]], axis=1)
    y = y * lax.rsqrt(dr0[...] + dr1[...] + 1.0)
    t = y + h[...]
    mean = jnp.mean(t, axis=1, keepdims=True)
    var = jnp.mean((t - mean) ** 2, axis=1, keepdims=True)
    hn = (t - mean) * lax.rsqrt(var + 1e-5) * lnsc[...] + lnof[...]
    h_out[...] = hn
    x = _mlp_block(hn, w0[...], b0[...], w1[...], b1[...])
    xs = x * lax.rsqrt(ds0[...] + ds1[...] + 1.0)
    xs0_out[...] = xs[:, :d2]
    xs1_out[...] = xs[:, d2:]


def _f2_body(n_valid, grid, a00, a01, a10, a11, xs0, xs1, h, dr0, dr1,
             lnsc, lnof, e2d, wd, bd, h_out, e_out, g_out, acc_ref):
    i = pl.program_id(0)
    y = jnp.concatenate([a00[...] + a01[...] + xs0[...],
                         a10[...] + a11[...] + xs1[...]], axis=1)
    y = y * lax.rsqrt(dr0[...] + dr1[...] + 1.0)
    t = y + h[...]
    mean = jnp.mean(t, axis=1, keepdims=True)
    var = jnp.mean((t - mean) ** 2, axis=1, keepdims=True)
    hn = (t - mean) * lax.rsqrt(var + 1e-5) * lnsc[...] + lnof[...]
    h_out[...] = hn
    e_out[...] = e2d[...] * 4.0

    blk = hn.shape[0]
    row = i * blk + lax.broadcasted_iota(jnp.int32, (blk, 1), 0)
    masked = jnp.where(row < n_valid, hn, 0.0)
    psum = jnp.sum(masked, axis=0, keepdims=True)

    @pl.when(i == 0)
    def _():
        acc_ref[...] = jnp.zeros_like(acc_ref)

    acc_ref[...] += psum

    @pl.when(i == grid - 1)
    def _():
        pooled = acc_ref[...] * (1.0 / n_valid)
        g_out[...] = jnp.dot(pooled, wd[...], precision=_PREC) + bd[...]


def kernel(nodes, edges, globals_, senders, receivers, W_embed, b_embed,
           W_mlp, b_mlp, ln_scale, ln_offset, W_dec, b_dec):
    n, d = nodes.shape
    e = senders.shape[0]
    latent = W_embed.shape[1]
    out_g = W_dec.shape[1]
    d2 = latent // 2

    n_acc = ((n + 1023) // 1024 + (0 if n % 1024 else 1)) * 1024
    if n_acc <= n:
        n_acc = n + 1024
    ndum = n_acc - n
    epad = ((e + NW * K - 1) // (NW * K)) * (NW * K)
    npad = epad - e
    drows = 2 * n_acc

    pad = (jnp.arange(npad, dtype=jnp.int32) % ndum) + n
    snd = jnp.concatenate([senders.astype(jnp.int32), pad])
    rcv = jnp.concatenate([receivers.astype(jnp.int32), pad])
    didx = jnp.concatenate([snd, rcv + n_acc])

    zer_half = jnp.zeros((n_acc, d2), jnp.float32)
    zer_deg = jnp.zeros((drows, 16), jnp.float32)
    ones16 = jnp.ones((K, 16), jnp.float32)

    deg_k = _make_degree(drows, 2 * epad)
    d0, d1 = deg_k(didx, ones16, zer_deg)
    ds0 = d0[:n_acc, 0:1]
    ds1 = d1[:n_acc, 0:1]
    dr0 = d0[n_acc:, 0:1]
    dr1 = d1[n_acc:, 0:1]

    nodes_pad = jnp.pad(nodes, ((0, n_acc - n), (0, 0)))
    be = b_embed.reshape(1, latent)
    bd = b_dec.reshape(1, out_g)

    grid = 16
    blk = n_acc // grid
    row_spec = pl.BlockSpec((blk, latent), lambda i: (i, 0))
    half_spec = pl.BlockSpec((blk, d2), lambda i: (i, 0))
    col_spec = pl.BlockSpec((blk, 1), lambda i: (i, 0))
    full_spec = pl.BlockSpec((d, latent), lambda i: (0, 0))
    vec_spec = pl.BlockSpec((1, latent), lambda i: (0, 0))

    half_shape = jax.ShapeDtypeStruct((n_acc, d2), jnp.float32)
    full_shape = jax.ShapeDtypeStruct((n_acc, latent), jnp.float32)

    f0 = pl.pallas_call(
        functools.partial(_f0_body, d2),
        grid=(grid,),
        in_specs=[row_spec, full_spec, vec_spec, full_spec, vec_spec,
                  full_spec, vec_spec, col_spec, col_spec],
        out_specs=[row_spec, half_spec, half_spec],
        out_shape=[full_shape, half_shape, half_shape],
    )
    h0, xs00, xs01 = f0(nodes_pad, W_embed, be,
                        W_mlp[0, 0], b_mlp[0, 0].reshape(1, -1),
                        W_mlp[0, 1], b_mlp[0, 1].reshape(1, -1), ds0, ds1)

    scat_k = _make_scatter(n_acc, epad, d2)

    a000, a001, a010, a011 = scat_k(xs00, xs01, snd, rcv, zer_half)
    f1 = pl.pallas_call(
        functools.partial(_f1_body, d2),
        grid=(grid,),
        in_specs=[half_spec, half_spec, half_spec, half_spec, half_spec,
                  half_spec, row_spec, col_spec, col_spec,
                  vec_spec, vec_spec, full_spec, vec_spec, full_spec,
                  vec_spec, col_spec, col_spec],
        out_specs=[row_spec, half_spec, half_spec],
        out_shape=[full_shape, half_shape, half_shape],
    )
    h1, xs10, xs11 = f1(a000, a001, a010, a011, xs00, xs01, h0, dr0, dr1,
                        ln_scale[0].reshape(1, -1), ln_offset[0].reshape(1, -1),
                        W_mlp[1, 0], b_mlp[1, 0].reshape(1, -1),
                        W_mlp[1, 1], b_mlp[1, 1].reshape(1, -1), ds0, ds1)

    a100, a101, a110, a111 = scat_k(xs10, xs11, snd, rcv, zer_half)

    e4r = e * edges.shape[1] // latent
    g2 = 10
    eblk = e4r // g2
    blk2 = n_acc // g2
    row2 = pl.BlockSpec((blk2, latent), lambda i: (i, 0))
    half2 = pl.BlockSpec((blk2, d2), lambda i: (i, 0))
    col2 = pl.BlockSpec((blk2, 1), lambda i: (i, 0))
    vec2 = pl.BlockSpec((1, latent), lambda i: (0, 0))
    e2d = edges.reshape(e4r, latent)

    f2 = pl.pallas_call(
        functools.partial(_f2_body, float(n), g2),
        grid=(g2,),
        in_specs=[half2, half2, half2, half2, half2, half2, row2,
                  col2, col2, vec2, vec2,
                  pl.BlockSpec((eblk, latent), lambda i: (i, 0)),
                  pl.BlockSpec((latent, out_g), lambda i: (0, 0)),
                  pl.BlockSpec((1, out_g), lambda i: (0, 0))],
        out_specs=[row2,
                   pl.BlockSpec((eblk, latent), lambda i: (i, 0)),
                   pl.BlockSpec((1, out_g), lambda i: (0, 0))],
        out_shape=[jax.ShapeDtypeStruct((n_acc, latent), jnp.float32),
                   jax.ShapeDtypeStruct((e4r, latent), jnp.float32),
                   jax.ShapeDtypeStruct((1, out_g), jnp.float32)],
        scratch_shapes=[pltpu.VMEM((1, latent), jnp.float32)],
    )
    h2, eout, g = f2(a100, a101, a110, a111, xs10, xs11, h1, dr0, dr1,
                     ln_scale[1].reshape(1, -1), ln_offset[1].reshape(1, -1),
                     e2d, W_dec, bd)

    return (h2[:n], eout.reshape(e, edges.shape[1]), g)


# speedup vs baseline: 6.2414x; 1.0448x over previous
import functools

import jax
import jax.numpy as jnp
from jax import lax
from jax.experimental import pallas as pl
from jax.experimental.pallas import tpu as pltpu
from jax.experimental.pallas import tpu_sc as plsc

NC = 2
NS = 16
NW = NC * NS
K = 512

_PREC = lax.Precision.DEFAULT
_SC_PARAMS = pltpu.CompilerParams(use_tc_tiling_on_sc=False)


def _make_scatter(n_acc, epad, d2):
    epw = epad // NW
    nch = epw // K
    rpw = n_acc // NS
    mesh = plsc.VectorSubcoreMesh(core_axis_name="c", subcore_axis_name="s")

    @functools.partial(
        pl.kernel,
        out_type=tuple(jax.ShapeDtypeStruct((n_acc, d2), jnp.float32)
                       for _ in range(4)),
        mesh=mesh,
        compiler_params=_SC_PARAMS,
        scratch_types=[
            pltpu.VMEM((K,), jnp.int32),
            pltpu.VMEM((K,), jnp.int32),
            pltpu.VMEM((K, d2), jnp.float32),
            pltpu.VMEM_SHARED((n_acc, d2), jnp.float32),
            pltpu.VMEM_SHARED((n_acc, d2), jnp.float32),
        ],
    )
    def scatter_k(xs0_hbm, xs1_hbm, snd_hbm, rcv_hbm, zer_hbm,
                  out00, out01, out10, out11,
                  idx_s, idx_r, rows, xs_sh, acc):
        cid = lax.axis_index("c")
        sid = lax.axis_index("s")
        wid = sid * NC + cid
        r0 = sid * rpw

        def one_half(xs_hbm, outA, outB):
            pltpu.sync_copy(xs_hbm.at[pl.ds(r0, rpw)], xs_sh.at[pl.ds(r0, rpw)])
            pltpu.sync_copy(zer_hbm.at[pl.ds(r0, rpw)], acc.at[pl.ds(r0, rpw)])
            plsc.subcore_barrier()

            def body(ch, carry):
                base = wid * epw + ch * K
                pltpu.sync_copy(snd_hbm.at[pl.ds(base, K)], idx_s)
                pltpu.sync_copy(rcv_hbm.at[pl.ds(base, K)], idx_r)
                pltpu.sync_copy(xs_sh.at[idx_s], rows)
                pltpu.sync_copy(rows, acc.at[idx_r], add=True)
                return carry

            lax.fori_loop(0, nch, body, 0)
            plsc.subcore_barrier()

            @pl.when(cid == 0)
            def _():
                pltpu.sync_copy(acc.at[pl.ds(r0, rpw)], outA.at[pl.ds(r0, rpw)])

            @pl.when(cid == 1)
            def _():
                pltpu.sync_copy(acc.at[pl.ds(r0, rpw)], outB.at[pl.ds(r0, rpw)])

            plsc.subcore_barrier()

        one_half(xs0_hbm, out00, out01)
        one_half(xs1_hbm, out10, out11)

    return scatter_k


def _make_degree(drows, m):
    mpw = m // NW
    nch = mpw // K
    rpw = drows // NS
    mesh = plsc.VectorSubcoreMesh(core_axis_name="c", subcore_axis_name="s")

    @functools.partial(
        pl.kernel,
        out_type=(jax.ShapeDtypeStruct((drows, 16), jnp.float32),
                  jax.ShapeDtypeStruct((drows, 16), jnp.float32)),
        mesh=mesh,
        compiler_params=_SC_PARAMS,
        scratch_types=[
            pltpu.VMEM((K,), jnp.int32),
            pltpu.VMEM((K, 16), jnp.float32),
            pltpu.VMEM_SHARED((drows, 16), jnp.float32),
        ],
    )
    def degree_k(didx_hbm, ones_hbm, zer_hbm, out0, out1, idx, ones_v, dacc):
        cid = lax.axis_index("c")
        sid = lax.axis_index("s")
        wid = sid * NC + cid
        r0 = sid * rpw
        pltpu.sync_copy(ones_hbm, ones_v)
        pltpu.sync_copy(zer_hbm.at[pl.ds(r0, rpw)], dacc.at[pl.ds(r0, rpw)])
        plsc.subcore_barrier()

        def body(ch, carry):
            base = wid * mpw + ch * K
            pltpu.sync_copy(didx_hbm.at[pl.ds(base, K)], idx)
            pltpu.sync_copy(ones_v, dacc.at[idx], add=True)
            return carry

        lax.fori_loop(0, nch, body, 0)
        plsc.subcore_barrier()

        @pl.when(cid == 0)
        def _():
            pltpu.sync_copy(dacc.at[pl.ds(r0, rpw)], out0.at[pl.ds(r0, rpw)])

        @pl.when(cid == 1)
        def _():
            pltpu.sync_copy(dacc.at[pl.ds(r0, rpw)], out1.at[pl.ds(r0, rpw)])

    return degree_k


def _mlp_block(x, w0, b0, w1, b1):
    x = jnp.maximum(jnp.dot(x, w0, precision=_PREC) + b0, 0.0)
    return jnp.maximum(jnp.dot(x, w1, precision=_PREC) + b1, 0.0)


def _f0_body(d2, nodes, we, be, w0, b0, w1, b1, ds0, ds1,
             h_out, xs0_out, xs1_out):
    h = jnp.dot(nodes[...], we[...], precision=_PREC) + be[...]
    h_out[...] = h
    x = _mlp_block(h, w0[...], b0[...], w1[...], b1[...])
    xs = x * lax.rsqrt(ds0[...] + ds1[...] + 1.0)
    xs0_out[...] = xs[:, :d2]
    xs1_out[...] = xs[:, d2:]


def _f1_body(d2, a00, a01, a10, a11, xs0, xs1, h, dr0, dr1, lnsc, lnof,
             w0, b0, w1, b1, ds0, ds1, h_out, xs0_out, xs1_out):
    y = jnp.concatenate([a00[...] + a01[...] + xs0[...],
                         a10[...] + a11[...] + xs1[...]], axis=1)
    y = y * lax.rsqrt(dr0[...] + dr1[...] + 1.0)
    t = y + h[...]
    mean = jnp.mean(t, axis=1, keepdims=True)
    var = jnp.mean((t - mean) ** 2, axis=1, keepdims=True)
    hn = (t - mean) * lax.rsqrt(var + 1e-5) * lnsc[...] + lnof[...]
    h_out[...] = hn
    x = _mlp_block(hn, w0[...], b0[...], w1[...], b1[...])
    xs = x * lax.rsqrt(ds0[...] + ds1[...] + 1.0)
    xs0_out[...] = xs[:, :d2]
    xs1_out[...] = xs[:, d2:]


def _f2_body(n_valid, grid, a00, a01, a10, a11, xs0, xs1, h, dr0, dr1,
             lnsc, lnof, e2d, wd, bd, h_out, e_out, g_out, acc_ref):
    i = pl.program_id(0)
    y = jnp.concatenate([a00[...] + a01[...] + xs0[...],
                         a10[...] + a11[...] + xs1[...]], axis=1)
    y = y * lax.rsqrt(dr0[...] + dr1[...] + 1.0)
    t = y + h[...]
    mean = jnp.mean(t, axis=1, keepdims=True)
    var = jnp.mean((t - mean) ** 2, axis=1, keepdims=True)
    hn = (t - mean) * lax.rsqrt(var + 1e-5) * lnsc[...] + lnof[...]
    h_out[...] = hn
    e_out[...] = e2d[...] * 4.0

    blk = hn.shape[0]
    row = i * blk + lax.broadcasted_iota(jnp.int32, (blk, 1), 0)
    masked = jnp.where(row < n_valid, hn, 0.0)
    psum = jnp.sum(masked, axis=0, keepdims=True)

    @pl.when(i == 0)
    def _():
        acc_ref[...] = jnp.zeros_like(acc_ref)

    acc_ref[...] += psum

    @pl.when(i == grid - 1)
    def _():
        pooled = acc_ref[...] * (1.0 / n_valid)
        g_out[...] = jnp.dot(pooled, wd[...], precision=_PREC) + bd[...]


def kernel(nodes, edges, globals_, senders, receivers, W_embed, b_embed,
           W_mlp, b_mlp, ln_scale, ln_offset, W_dec, b_dec):
    n, d = nodes.shape
    e = senders.shape[0]
    latent = W_embed.shape[1]
    out_g = W_dec.shape[1]
    d2 = latent // 2

    n_acc = ((n + 1023) // 1024 + (0 if n % 1024 else 1)) * 1024
    if n_acc <= n:
        n_acc = n + 1024
    ndum = n_acc - n
    epad = ((e + NW * K - 1) // (NW * K)) * (NW * K)
    npad = epad - e
    drows = 2 * n_acc

    pad = (jnp.arange(npad, dtype=jnp.int32) % ndum) + n
    snd = jnp.concatenate([senders.astype(jnp.int32), pad])
    rcv = jnp.concatenate([receivers.astype(jnp.int32), pad])
    didx = jnp.concatenate([snd, rcv + n_acc])

    zer_half = jnp.zeros((n_acc, d2), jnp.float32)
    zer_deg = jnp.zeros((drows, 16), jnp.float32)
    ones16 = jnp.ones((K, 16), jnp.float32)

    deg_k = _make_degree(drows, 2 * epad)
    d0, d1 = deg_k(didx, ones16, zer_deg)
    ds0 = d0[:n_acc, 0:1]
    ds1 = d1[:n_acc, 0:1]
    dr0 = d0[n_acc:, 0:1]
    dr1 = d1[n_acc:, 0:1]

    nodes_pad = jnp.pad(nodes, ((0, n_acc - n), (0, 0)))
    be = b_embed.reshape(1, latent)
    bd = b_dec.reshape(1, out_g)

    grid = 16
    blk = n_acc // grid
    row_spec = pl.BlockSpec((blk, latent), lambda i: (i, 0))
    half_spec = pl.BlockSpec((blk, d2), lambda i: (i, 0))
    col_spec = pl.BlockSpec((blk, 1), lambda i: (i, 0))
    full_spec = pl.BlockSpec((d, latent), lambda i: (0, 0))
    vec_spec = pl.BlockSpec((1, latent), lambda i: (0, 0))

    half_shape = jax.ShapeDtypeStruct((n_acc, d2), jnp.float32)
    full_shape = jax.ShapeDtypeStruct((n_acc, latent), jnp.float32)

    f0 = pl.pallas_call(
        functools.partial(_f0_body, d2),
        grid=(grid,),
        in_specs=[row_spec, full_spec, vec_spec, full_spec, vec_spec,
                  full_spec, vec_spec, col_spec, col_spec],
        out_specs=[row_spec, half_spec, half_spec],
        out_shape=[full_shape, half_shape, half_shape],
    )
    h0, xs00, xs01 = f0(nodes_pad, W_embed, be,
                        W_mlp[0, 0], b_mlp[0, 0].reshape(1, -1),
                        W_mlp[0, 1], b_mlp[0, 1].reshape(1, -1), ds0, ds1)

    scat_k = _make_scatter(n_acc, epad, d2)

    a000, a001, a010, a011 = scat_k(xs00, xs01, snd, rcv, zer_half)
    f1 = pl.pallas_call(
        functools.partial(_f1_body, d2),
        grid=(grid,),
        in_specs=[half_spec, half_spec, half_spec, half_spec, half_spec,
                  half_spec, row_spec, col_spec, col_spec,
                  vec_spec, vec_spec, full_spec, vec_spec, full_spec,
                  vec_spec, col_spec, col_spec],
        out_specs=[row_spec, half_spec, half_spec],
        out_shape=[full_shape, half_shape, half_shape],
    )
    h1, xs10, xs11 = f1(a000, a001, a010, a011, xs00, xs01, h0, dr0, dr1,
                        ln_scale[0].reshape(1, -1), ln_offset[0].reshape(1, -1),
                        W_mlp[1, 0], b_mlp[1, 0].reshape(1, -1),
                        W_mlp[1, 1], b_mlp[1, 1].reshape(1, -1), ds0, ds1)

    a100, a101, a110, a111 = scat_k(xs10, xs11, snd, rcv, zer_half)

    e4r = e * edges.shape[1] // latent
    g2 = 10
    eblk = e4r // g2
    blk2 = n_acc // g2
    row2 = pl.BlockSpec((blk2, latent), lambda i: (i, 0))
    half2 = pl.BlockSpec((blk2, d2), lambda i: (i, 0))
    col2 = pl.BlockSpec((blk2, 1), lambda i: (i, 0))
    vec2 = pl.BlockSpec((1, latent), lambda i: (0, 0))
    e2d = edges.reshape(e4r, latent)

    f2 = pl.pallas_call(
        functools.partial(_f2_body, float(n), g2),
        grid=(g2,),
        in_specs=[half2, half2, half2, half2, half2, half2, row2,
                  col2, col2, vec2, vec2,
                  pl.BlockSpec((eblk, latent), lambda i: (i, 0)),
                  pl.BlockSpec((latent, out_g), lambda i: (0, 0)),
                  pl.BlockSpec((1, out_g), lambda i: (0, 0))],
        out_specs=[row2,
                   pl.BlockSpec((eblk, latent), lambda i: (i, 0)),
                   pl.BlockSpec((1, out_g), lambda i: (0, 0))],
        out_shape=[jax.ShapeDtypeStruct((n_acc, latent), jnp.float32),
                   jax.ShapeDtypeStruct((e4r, latent), jnp.float32),
                   jax.ShapeDtypeStruct((1, out_g), jnp.float32)],
        scratch_shapes=[pltpu.VMEM((1, latent), jnp.float32)],
    )
    h2, eout, g = f2(a100, a101, a110, a111, xs10, xs11, h1, dr0, dr1,
                     ln_scale[1].reshape(1, -1), ln_offset[1].reshape(1, -1),
                     e2d, W_dec, bd)

    return (h2[:n], eout.reshape(e, edges.shape[1]), g)


# speedup vs baseline: 6.3077x; 1.0106x over previous
import functools

import jax
import jax.numpy as jnp
from jax import lax
from jax.experimental import pallas as pl
from jax.experimental.pallas import tpu as pltpu
from jax.experimental.pallas import tpu_sc as plsc

NC = 2
NS = 16
NW = NC * NS
K = 512

_PREC = lax.Precision.DEFAULT
_SC_PARAMS = pltpu.CompilerParams(use_tc_tiling_on_sc=False)


def _make_scatter(n_acc, epad, d2):
    epw = epad // NW
    nch = epw // K
    rpw = n_acc // NS
    mesh = plsc.VectorSubcoreMesh(core_axis_name="c", subcore_axis_name="s")

    @functools.partial(
        pl.kernel,
        out_type=tuple(jax.ShapeDtypeStruct((n_acc, d2), jnp.float32)
                       for _ in range(4)),
        mesh=mesh,
        compiler_params=_SC_PARAMS,
        scratch_types=[
            pltpu.VMEM((K,), jnp.int32),
            pltpu.VMEM((K,), jnp.int32),
            pltpu.VMEM((K, d2), jnp.float32),
            pltpu.VMEM_SHARED((n_acc, d2), jnp.float32),
            pltpu.VMEM_SHARED((n_acc, d2), jnp.float32),
        ],
    )
    def scatter_k(xs0_hbm, xs1_hbm, snd_hbm, rcv_hbm, zer_hbm,
                  out00, out01, out10, out11,
                  idx_s, idx_r, rows, xs_sh, acc):
        cid = lax.axis_index("c")
        sid = lax.axis_index("s")
        wid = sid * NC + cid
        r0 = sid * rpw

        def one_half(xs_hbm, outA, outB):
            pltpu.sync_copy(xs_hbm.at[pl.ds(r0, rpw)], xs_sh.at[pl.ds(r0, rpw)])
            pltpu.sync_copy(zer_hbm.at[pl.ds(r0, rpw)], acc.at[pl.ds(r0, rpw)])
            plsc.subcore_barrier()

            def body(ch, carry):
                base = wid * epw + ch * K
                pltpu.sync_copy(snd_hbm.at[pl.ds(base, K)], idx_s)
                pltpu.sync_copy(rcv_hbm.at[pl.ds(base, K)], idx_r)
                pltpu.sync_copy(xs_sh.at[idx_s], rows)
                pltpu.sync_copy(rows, acc.at[idx_r], add=True)
                return carry

            lax.fori_loop(0, nch, body, 0)
            plsc.subcore_barrier()

            @pl.when(cid == 0)
            def _():
                pltpu.sync_copy(acc.at[pl.ds(r0, rpw)], outA.at[pl.ds(r0, rpw)])

            @pl.when(cid == 1)
            def _():
                pltpu.sync_copy(acc.at[pl.ds(r0, rpw)], outB.at[pl.ds(r0, rpw)])

            plsc.subcore_barrier()

        one_half(xs0_hbm, out00, out01)
        one_half(xs1_hbm, out10, out11)

    return scatter_k


def _make_degree(drows, m):
    mpw = m // NW
    nch = mpw // K
    rpw = drows // NS
    mesh = plsc.VectorSubcoreMesh(core_axis_name="c", subcore_axis_name="s")

    @functools.partial(
        pl.kernel,
        out_type=(jax.ShapeDtypeStruct((drows, 16), jnp.float32),
                  jax.ShapeDtypeStruct((drows, 16), jnp.float32)),
        mesh=mesh,
        compiler_params=_SC_PARAMS,
        scratch_types=[
            pltpu.VMEM((K,), jnp.int32),
            pltpu.VMEM((K, 16), jnp.float32),
            pltpu.VMEM_SHARED((drows, 16), jnp.float32),
        ],
    )
    def degree_k(didx_hbm, ones_hbm, zer_hbm, out0, out1, idx, ones_v, dacc):
        cid = lax.axis_index("c")
        sid = lax.axis_index("s")
        wid = sid * NC + cid
        r0 = sid * rpw
        pltpu.sync_copy(ones_hbm, ones_v)
        pltpu.sync_copy(zer_hbm.at[pl.ds(r0, rpw)], dacc.at[pl.ds(r0, rpw)])
        plsc.subcore_barrier()

        def body(ch, carry):
            base = wid * mpw + ch * K
            pltpu.sync_copy(didx_hbm.at[pl.ds(base, K)], idx)
            pltpu.sync_copy(ones_v, dacc.at[idx], add=True)
            return carry

        lax.fori_loop(0, nch, body, 0)
        plsc.subcore_barrier()

        @pl.when(cid == 0)
        def _():
            pltpu.sync_copy(dacc.at[pl.ds(r0, rpw)], out0.at[pl.ds(r0, rpw)])

        @pl.when(cid == 1)
        def _():
            pltpu.sync_copy(dacc.at[pl.ds(r0, rpw)], out1.at[pl.ds(r0, rpw)])

    return degree_k


def _mlp_block(x, w0, b0, w1, b1):
    x = jnp.maximum(jnp.dot(x, w0, precision=_PREC) + b0, 0.0)
    return jnp.maximum(jnp.dot(x, w1, precision=_PREC) + b1, 0.0)


def _f0_body(d2, nodes, we, be, w0, b0, w1, b1, ds0, ds1,
             h_out, xs0_out, xs1_out):
    h = jnp.dot(nodes[...], we[...], precision=_PREC) + be[...]
    h_out[...] = h
    x = _mlp_block(h, w0[...], b0[...], w1[...], b1[...])
    xs = x * lax.rsqrt(ds0[...] + ds1[...] + 1.0)
    xs0_out[...] = xs[:, :d2]
    xs1_out[...] = xs[:, d2:]


def _f1_body(d2, a00, a01, a10, a11, xs0, xs1, h, dr0, dr1, lnsc, lnof,
             w0, b0, w1, b1, ds0, ds1, h_out, xs0_out, xs1_out):
    y = jnp.concatenate([a00[...] + a01[...] + xs0[...],
                         a10[...] + a11[...] + xs1[...]], axis=1)
    y = y * lax.rsqrt(dr0[...] + dr1[...] + 1.0)
    t = y + h[...]
    mean = jnp.mean(t, axis=1, keepdims=True)
    var = jnp.mean((t - mean) ** 2, axis=1, keepdims=True)
    hn = (t - mean) * lax.rsqrt(var + 1e-5) * lnsc[...] + lnof[...]
    h_out[...] = hn
    x = _mlp_block(hn, w0[...], b0[...], w1[...], b1[...])
    xs = x * lax.rsqrt(ds0[...] + ds1[...] + 1.0)
    xs0_out[...] = xs[:, :d2]
    xs1_out[...] = xs[:, d2:]


def _f2_body(n_valid, grid, a00, a01, a10, a11, xs0, xs1, h, dr0, dr1,
             lnsc, lnof, e2d, wd, bd, h_out, e_out, g_out, acc_ref):
    i = pl.program_id(0)
    y = jnp.concatenate([a00[...] + a01[...] + xs0[...],
                         a10[...] + a11[...] + xs1[...]], axis=1)
    y = y * lax.rsqrt(dr0[...] + dr1[...] + 1.0)
    t = y + h[...]
    mean = jnp.mean(t, axis=1, keepdims=True)
    var = jnp.mean((t - mean) ** 2, axis=1, keepdims=True)
    hn = (t - mean) * lax.rsqrt(var + 1e-5) * lnsc[...] + lnof[...]
    h_out[...] = hn
    e_out[...] = e2d[...] * 4.0

    blk = hn.shape[0]
    row = i * blk + lax.broadcasted_iota(jnp.int32, (blk, 1), 0)
    masked = jnp.where(row < n_valid, hn, 0.0)
    psum = jnp.sum(masked, axis=0, keepdims=True)

    @pl.when(i == 0)
    def _():
        acc_ref[...] = jnp.zeros_like(acc_ref)

    acc_ref[...] += psum

    @pl.when(i == grid - 1)
    def _():
        pooled = acc_ref[...] * (1.0 / n_valid)
        g_out[...] = jnp.dot(pooled, wd[...], precision=_PREC) + bd[...]


def kernel(nodes, edges, globals_, senders, receivers, W_embed, b_embed,
           W_mlp, b_mlp, ln_scale, ln_offset, W_dec, b_dec):
    n, d = nodes.shape
    e = senders.shape[0]
    latent = W_embed.shape[1]
    out_g = W_dec.shape[1]
    d2 = latent // 2

    n_acc = ((n + 1023) // 1024 + (0 if n % 1024 else 1)) * 1024
    if n_acc <= n:
        n_acc = n + 1024
    ndum = n_acc - n
    epad = ((e + NW * K - 1) // (NW * K)) * (NW * K)
    npad = epad - e
    drows = 2 * n_acc

    pad = (jnp.arange(npad, dtype=jnp.int32) % ndum) + n
    snd = jnp.concatenate([senders.astype(jnp.int32), pad])
    rcv = jnp.concatenate([receivers.astype(jnp.int32), pad])
    didx = jnp.concatenate([snd, rcv + n_acc])

    zer_half = jnp.zeros((n_acc, d2), jnp.float32)
    zer_deg = jnp.zeros((drows, 16), jnp.float32)
    ones16 = jnp.ones((K, 16), jnp.float32)

    deg_k = _make_degree(drows, 2 * epad)
    d0, d1 = deg_k(didx, ones16, zer_deg)
    ds0 = d0[:n_acc, 0:1]
    ds1 = d1[:n_acc, 0:1]
    dr0 = d0[n_acc:, 0:1]
    dr1 = d1[n_acc:, 0:1]

    nodes_pad = jnp.pad(nodes, ((0, n_acc - n), (0, 0)))
    be = b_embed.reshape(1, latent)
    bd = b_dec.reshape(1, out_g)

    grid = 8
    blk = n_acc // grid
    row_spec = pl.BlockSpec((blk, latent), lambda i: (i, 0))
    half_spec = pl.BlockSpec((blk, d2), lambda i: (i, 0))
    col_spec = pl.BlockSpec((blk, 1), lambda i: (i, 0))
    full_spec = pl.BlockSpec((d, latent), lambda i: (0, 0))
    vec_spec = pl.BlockSpec((1, latent), lambda i: (0, 0))

    half_shape = jax.ShapeDtypeStruct((n_acc, d2), jnp.float32)
    full_shape = jax.ShapeDtypeStruct((n_acc, latent), jnp.float32)

    f0 = pl.pallas_call(
        functools.partial(_f0_body, d2),
        grid=(grid,),
        in_specs=[row_spec, full_spec, vec_spec, full_spec, vec_spec,
                  full_spec, vec_spec, col_spec, col_spec],
        out_specs=[row_spec, half_spec, half_spec],
        out_shape=[full_shape, half_shape, half_shape],
    )
    h0, xs00, xs01 = f0(nodes_pad, W_embed, be,
                        W_mlp[0, 0], b_mlp[0, 0].reshape(1, -1),
                        W_mlp[0, 1], b_mlp[0, 1].reshape(1, -1), ds0, ds1)

    scat_k = _make_scatter(n_acc, epad, d2)

    a000, a001, a010, a011 = scat_k(xs00, xs01, snd, rcv, zer_half)
    f1 = pl.pallas_call(
        functools.partial(_f1_body, d2),
        grid=(grid,),
        in_specs=[half_spec, half_spec, half_spec, half_spec, half_spec,
                  half_spec, row_spec, col_spec, col_spec,
                  vec_spec, vec_spec, full_spec, vec_spec, full_spec,
                  vec_spec, col_spec, col_spec],
        out_specs=[row_spec, half_spec, half_spec],
        out_shape=[full_shape, half_shape, half_shape],
    )
    h1, xs10, xs11 = f1(a000, a001, a010, a011, xs00, xs01, h0, dr0, dr1,
                        ln_scale[0].reshape(1, -1), ln_offset[0].reshape(1, -1),
                        W_mlp[1, 0], b_mlp[1, 0].reshape(1, -1),
                        W_mlp[1, 1], b_mlp[1, 1].reshape(1, -1), ds0, ds1)

    a100, a101, a110, a111 = scat_k(xs10, xs11, snd, rcv, zer_half)

    e4r = e * edges.shape[1] // latent
    g2 = 5
    eblk = e4r // g2
    blk2 = n_acc // g2
    row2 = pl.BlockSpec((blk2, latent), lambda i: (i, 0))
    half2 = pl.BlockSpec((blk2, d2), lambda i: (i, 0))
    col2 = pl.BlockSpec((blk2, 1), lambda i: (i, 0))
    vec2 = pl.BlockSpec((1, latent), lambda i: (0, 0))
    e2d = edges.reshape(e4r, latent)

    f2 = pl.pallas_call(
        functools.partial(_f2_body, float(n), g2),
        grid=(g2,),
        in_specs=[half2, half2, half2, half2, half2, half2, row2,
                  col2, col2, vec2, vec2,
                  pl.BlockSpec((eblk, latent), lambda i: (i, 0)),
                  pl.BlockSpec((latent, out_g), lambda i: (0, 0)),
                  pl.BlockSpec((1, out_g), lambda i: (0, 0))],
        out_specs=[row2,
                   pl.BlockSpec((eblk, latent), lambda i: (i, 0)),
                   pl.BlockSpec((1, out_g), lambda i: (0, 0))],
        out_shape=[jax.ShapeDtypeStruct((n_acc, latent), jnp.float32),
                   jax.ShapeDtypeStruct((e4r, latent), jnp.float32),
                   jax.ShapeDtypeStruct((1, out_g), jnp.float32)],
        scratch_shapes=[pltpu.VMEM((1, latent), jnp.float32)],
    )
    h2, eout, g = f2(a100, a101, a110, a111, xs10, xs11, h1, dr0, dr1,
                     ln_scale[1].reshape(1, -1), ln_offset[1].reshape(1, -1),
                     e2d, W_dec, bd)

    return (h2[:n], eout.reshape(e, edges.shape[1]), g)


# speedup vs baseline: 6.4174x; 1.0174x over previous
import functools

import jax
import jax.numpy as jnp
from jax import lax
from jax.experimental import pallas as pl
from jax.experimental.pallas import tpu as pltpu
from jax.experimental.pallas import tpu_sc as plsc

NC = 2
NS = 16
NW = NC * NS
K = 640

_PREC = lax.Precision.DEFAULT
_SC_PARAMS = pltpu.CompilerParams(use_tc_tiling_on_sc=False)


def _make_scatter(n_acc, epad, d2):
    epw = epad // NW
    nch = epw // K
    rpw = n_acc // NS
    mesh = plsc.VectorSubcoreMesh(core_axis_name="c", subcore_axis_name="s")

    @functools.partial(
        pl.kernel,
        out_type=tuple(jax.ShapeDtypeStruct((n_acc, d2), jnp.float32)
                       for _ in range(4)),
        mesh=mesh,
        compiler_params=_SC_PARAMS,
        scratch_types=[
            pltpu.VMEM((K,), jnp.int32),
            pltpu.VMEM((K,), jnp.int32),
            pltpu.VMEM((K, d2), jnp.float32),
            pltpu.VMEM_SHARED((n_acc, d2), jnp.float32),
            pltpu.VMEM_SHARED((n_acc, d2), jnp.float32),
        ],
    )
    def scatter_k(xs0_hbm, xs1_hbm, snd_hbm, rcv_hbm, zer_hbm,
                  out00, out01, out10, out11,
                  idx_s, idx_r, rows, xs_sh, acc):
        cid = lax.axis_index("c")
        sid = lax.axis_index("s")
        wid = sid * NC + cid
        r0 = sid * rpw

        def one_half(xs_hbm, outA, outB):
            pltpu.sync_copy(xs_hbm.at[pl.ds(r0, rpw)], xs_sh.at[pl.ds(r0, rpw)])
            pltpu.sync_copy(zer_hbm.at[pl.ds(r0, rpw)], acc.at[pl.ds(r0, rpw)])
            plsc.subcore_barrier()

            def body(ch, carry):
                base = wid * epw + ch * K
                pltpu.sync_copy(snd_hbm.at[pl.ds(base, K)], idx_s)
                pltpu.sync_copy(rcv_hbm.at[pl.ds(base, K)], idx_r)
                pltpu.sync_copy(xs_sh.at[idx_s], rows)
                pltpu.sync_copy(rows, acc.at[idx_r], add=True)
                return carry

            lax.fori_loop(0, nch, body, 0)
            plsc.subcore_barrier()

            @pl.when(cid == 0)
            def _():
                pltpu.sync_copy(acc.at[pl.ds(r0, rpw)], outA.at[pl.ds(r0, rpw)])

            @pl.when(cid == 1)
            def _():
                pltpu.sync_copy(acc.at[pl.ds(r0, rpw)], outB.at[pl.ds(r0, rpw)])

            plsc.subcore_barrier()

        one_half(xs0_hbm, out00, out01)
        one_half(xs1_hbm, out10, out11)

    return scatter_k


def _make_degree(drows, m):
    mpw = m // NW
    nch = mpw // K
    rpw = drows // NS
    mesh = plsc.VectorSubcoreMesh(core_axis_name="c", subcore_axis_name="s")

    @functools.partial(
        pl.kernel,
        out_type=(jax.ShapeDtypeStruct((drows, 16), jnp.float32),
                  jax.ShapeDtypeStruct((drows, 16), jnp.float32)),
        mesh=mesh,
        compiler_params=_SC_PARAMS,
        scratch_types=[
            pltpu.VMEM((K,), jnp.int32),
            pltpu.VMEM((K, 16), jnp.float32),
            pltpu.VMEM_SHARED((drows, 16), jnp.float32),
        ],
    )
    def degree_k(didx_hbm, ones_hbm, zer_hbm, out0, out1, idx, ones_v, dacc):
        cid = lax.axis_index("c")
        sid = lax.axis_index("s")
        wid = sid * NC + cid
        r0 = sid * rpw
        pltpu.sync_copy(ones_hbm, ones_v)
        pltpu.sync_copy(zer_hbm.at[pl.ds(r0, rpw)], dacc.at[pl.ds(r0, rpw)])
        plsc.subcore_barrier()

        def body(ch, carry):
            base = wid * mpw + ch * K
            pltpu.sync_copy(didx_hbm.at[pl.ds(base, K)], idx)
            pltpu.sync_copy(ones_v, dacc.at[idx], add=True)
            return carry

        lax.fori_loop(0, nch, body, 0)
        plsc.subcore_barrier()

        @pl.when(cid == 0)
        def _():
            pltpu.sync_copy(dacc.at[pl.ds(r0, rpw)], out0.at[pl.ds(r0, rpw)])

        @pl.when(cid == 1)
        def _():
            pltpu.sync_copy(dacc.at[pl.ds(r0, rpw)], out1.at[pl.ds(r0, rpw)])

    return degree_k


def _mlp_block(x, w0, b0, w1, b1):
    x = jnp.maximum(jnp.dot(x, w0, precision=_PREC) + b0, 0.0)
    return jnp.maximum(jnp.dot(x, w1, precision=_PREC) + b1, 0.0)


def _f0_body(d2, nodes, we, be, w0, b0, w1, b1, ds0, ds1,
             h_out, xs0_out, xs1_out):
    h = jnp.dot(nodes[...], we[...], precision=_PREC) + be[...]
    h_out[...] = h
    x = _mlp_block(h, w0[...], b0[...], w1[...], b1[...])
    xs = x * lax.rsqrt(ds0[...] + ds1[...] + 1.0)
    xs0_out[...] = xs[:, :d2]
    xs1_out[...] = xs[:, d2:]


def _f1_body(d2, a00, a01, a10, a11, xs0, xs1, h, dr0, dr1, lnsc, lnof,
             w0, b0, w1, b1, ds0, ds1, h_out, xs0_out, xs1_out):
    y = jnp.concatenate([a00[...] + a01[...] + xs0[...],
                         a10[...] + a11[...] + xs1[...]], axis=1)
    y = y * lax.rsqrt(dr0[...] + dr1[...] + 1.0)
    t = y + h[...]
    mean = jnp.mean(t, axis=1, keepdims=True)
    var = jnp.mean((t - mean) ** 2, axis=1, keepdims=True)
    hn = (t - mean) * lax.rsqrt(var + 1e-5) * lnsc[...] + lnof[...]
    h_out[...] = hn
    x = _mlp_block(hn, w0[...], b0[...], w1[...], b1[...])
    xs = x * lax.rsqrt(ds0[...] + ds1[...] + 1.0)
    xs0_out[...] = xs[:, :d2]
    xs1_out[...] = xs[:, d2:]


def _f2_body(n_valid, grid, a00, a01, a10, a11, xs0, xs1, h, dr0, dr1,
             lnsc, lnof, e2d, wd, bd, h_out, e_out, g_out, acc_ref):
    i = pl.program_id(0)
    y = jnp.concatenate([a00[...] + a01[...] + xs0[...],
                         a10[...] + a11[...] + xs1[...]], axis=1)
    y = y * lax.rsqrt(dr0[...] + dr1[...] + 1.0)
    t = y + h[...]
    mean = jnp.mean(t, axis=1, keepdims=True)
    var = jnp.mean((t - mean) ** 2, axis=1, keepdims=True)
    hn = (t - mean) * lax.rsqrt(var + 1e-5) * lnsc[...] + lnof[...]
    h_out[...] = hn
    e_out[...] = e2d[...] * 4.0

    blk = hn.shape[0]
    row = i * blk + lax.broadcasted_iota(jnp.int32, (blk, 1), 0)
    masked = jnp.where(row < n_valid, hn, 0.0)
    psum = jnp.sum(masked, axis=0, keepdims=True)

    @pl.when(i == 0)
    def _():
        acc_ref[...] = jnp.zeros_like(acc_ref)

    acc_ref[...] += psum

    @pl.when(i == grid - 1)
    def _():
        pooled = acc_ref[...] * (1.0 / n_valid)
        g_out[...] = jnp.dot(pooled, wd[...], precision=_PREC) + bd[...]


def kernel(nodes, edges, globals_, senders, receivers, W_embed, b_embed,
           W_mlp, b_mlp, ln_scale, ln_offset, W_dec, b_dec):
    n, d = nodes.shape
    e = senders.shape[0]
    latent = W_embed.shape[1]
    out_g = W_dec.shape[1]
    d2 = latent // 2

    n_acc = ((n + 1023) // 1024 + (0 if n % 1024 else 1)) * 1024
    if n_acc <= n:
        n_acc = n + 1024
    ndum = n_acc - n
    epad = ((e + NW * K - 1) // (NW * K)) * (NW * K)
    npad = epad - e
    drows = 2 * n_acc

    pad = (jnp.arange(npad, dtype=jnp.int32) % ndum) + n
    snd = jnp.concatenate([senders.astype(jnp.int32), pad])
    rcv = jnp.concatenate([receivers.astype(jnp.int32), pad])
    didx = jnp.concatenate([snd, rcv + n_acc])

    zer_half = jnp.zeros((n_acc, d2), jnp.float32)
    zer_deg = jnp.zeros((drows, 16), jnp.float32)
    ones16 = jnp.ones((K, 16), jnp.float32)

    deg_k = _make_degree(drows, 2 * epad)
    d0, d1 = deg_k(didx, ones16, zer_deg)
    ds0 = d0[:n_acc, 0:1]
    ds1 = d1[:n_acc, 0:1]
    dr0 = d0[n_acc:, 0:1]
    dr1 = d1[n_acc:, 0:1]

    nodes_pad = jnp.pad(nodes, ((0, n_acc - n), (0, 0)))
    be = b_embed.reshape(1, latent)
    bd = b_dec.reshape(1, out_g)

    grid = 8
    blk = n_acc // grid
    row_spec = pl.BlockSpec((blk, latent), lambda i: (i, 0))
    half_spec = pl.BlockSpec((blk, d2), lambda i: (i, 0))
    col_spec = pl.BlockSpec((blk, 1), lambda i: (i, 0))
    full_spec = pl.BlockSpec((d, latent), lambda i: (0, 0))
    vec_spec = pl.BlockSpec((1, latent), lambda i: (0, 0))

    half_shape = jax.ShapeDtypeStruct((n_acc, d2), jnp.float32)
    full_shape = jax.ShapeDtypeStruct((n_acc, latent), jnp.float32)

    f0 = pl.pallas_call(
        functools.partial(_f0_body, d2),
        grid=(grid,),
        in_specs=[row_spec, full_spec, vec_spec, full_spec, vec_spec,
                  full_spec, vec_spec, col_spec, col_spec],
        out_specs=[row_spec, half_spec, half_spec],
        out_shape=[full_shape, half_shape, half_shape],
    )
    h0, xs00, xs01 = f0(nodes_pad, W_embed, be,
                        W_mlp[0, 0], b_mlp[0, 0].reshape(1, -1),
                        W_mlp[0, 1], b_mlp[0, 1].reshape(1, -1), ds0, ds1)

    scat_k = _make_scatter(n_acc, epad, d2)

    a000, a001, a010, a011 = scat_k(xs00, xs01, snd, rcv, zer_half)
    f1 = pl.pallas_call(
        functools.partial(_f1_body, d2),
        grid=(grid,),
        in_specs=[half_spec, half_spec, half_spec, half_spec, half_spec,
                  half_spec, row_spec, col_spec, col_spec,
                  vec_spec, vec_spec, full_spec, vec_spec, full_spec,
                  vec_spec, col_spec, col_spec],
        out_specs=[row_spec, half_spec, half_spec],
        out_shape=[full_shape, half_shape, half_shape],
    )
    h1, xs10, xs11 = f1(a000, a001, a010, a011, xs00, xs01, h0, dr0, dr1,
                        ln_scale[0].reshape(1, -1), ln_offset[0].reshape(1, -1),
                        W_mlp[1, 0], b_mlp[1, 0].reshape(1, -1),
                        W_mlp[1, 1], b_mlp[1, 1].reshape(1, -1), ds0, ds1)

    a100, a101, a110, a111 = scat_k(xs10, xs11, snd, rcv, zer_half)

    e4r = e * edges.shape[1] // latent
    g2 = 5
    eblk = e4r // g2
    blk2 = n_acc // g2
    row2 = pl.BlockSpec((blk2, latent), lambda i: (i, 0))
    half2 = pl.BlockSpec((blk2, d2), lambda i: (i, 0))
    col2 = pl.BlockSpec((blk2, 1), lambda i: (i, 0))
    vec2 = pl.BlockSpec((1, latent), lambda i: (0, 0))
    e2d = edges.reshape(e4r, latent)

    f2 = pl.pallas_call(
        functools.partial(_f2_body, float(n), g2),
        grid=(g2,),
        in_specs=[half2, half2, half2, half2, half2, half2, row2,
                  col2, col2, vec2, vec2,
                  pl.BlockSpec((eblk, latent), lambda i: (i, 0)),
                  pl.BlockSpec((latent, out_g), lambda i: (0, 0)),
                  pl.BlockSpec((1, out_g), lambda i: (0, 0))],
        out_specs=[row2,
                   pl.BlockSpec((eblk, latent), lambda i: (i, 0)),
                   pl.BlockSpec((1, out_g), lambda i: (0, 0))],
        out_shape=[jax.ShapeDtypeStruct((n_acc, latent), jnp.float32),
                   jax.ShapeDtypeStruct((e4r, latent), jnp.float32),
                   jax.ShapeDtypeStruct((1, out_g), jnp.float32)],
        scratch_shapes=[pltpu.VMEM((1, latent), jnp.float32)],
    )
    h2, eout, g = f2(a100, a101, a110, a111, xs10, xs11, h1, dr0, dr1,
                     ln_scale[1].reshape(1, -1), ln_offset[1].reshape(1, -1),
                     e2d, W_dec, bd)

    return (h2[:n], eout.reshape(e, edges.shape[1]), g)


# speedup vs baseline: 6.4818x; 1.0100x over previous
import functools

import jax
import jax.numpy as jnp
from jax import lax
from jax.experimental import pallas as pl
from jax.experimental.pallas import tpu as pltpu
from jax.experimental.pallas import tpu_sc as plsc

NC = 2
NS = 16
NW = NC * NS
K = 640

_PREC = lax.Precision.DEFAULT
_SC_PARAMS = pltpu.CompilerParams(use_tc_tiling_on_sc=False)


def _make_scatter(n_acc, epad, d2):
    epw = epad // NW
    nch = epw // K
    rpw = n_acc // NS
    mesh = plsc.VectorSubcoreMesh(core_axis_name="c", subcore_axis_name="s")

    @functools.partial(
        pl.kernel,
        out_type=tuple(jax.ShapeDtypeStruct((n_acc, d2), jnp.float32)
                       for _ in range(4)),
        mesh=mesh,
        compiler_params=_SC_PARAMS,
        scratch_types=[
            pltpu.VMEM((K,), jnp.int32),
            pltpu.VMEM((K,), jnp.int32),
            pltpu.VMEM((K, d2), jnp.float32),
            pltpu.VMEM_SHARED((n_acc, d2), jnp.float32),
            pltpu.VMEM_SHARED((n_acc, d2), jnp.float32),
        ],
    )
    def scatter_k(xs0_hbm, xs1_hbm, snd_hbm, rcv_hbm, zer_hbm,
                  out00, out01, out10, out11,
                  idx_s, idx_r, rows, xs_sh, acc):
        cid = lax.axis_index("c")
        sid = lax.axis_index("s")
        wid = sid * NC + cid
        r0 = sid * rpw

        def one_half(xs_hbm, outA, outB):
            pltpu.sync_copy(xs_hbm.at[pl.ds(r0, rpw)], xs_sh.at[pl.ds(r0, rpw)])
            pltpu.sync_copy(zer_hbm.at[pl.ds(r0, rpw)], acc.at[pl.ds(r0, rpw)])
            plsc.subcore_barrier()

            def body(ch, carry):
                base = wid * epw + ch * K
                pltpu.sync_copy(snd_hbm.at[pl.ds(base, K)], idx_s)
                pltpu.sync_copy(rcv_hbm.at[pl.ds(base, K)], idx_r)
                pltpu.sync_copy(xs_sh.at[idx_s], rows)
                pltpu.sync_copy(rows, acc.at[idx_r], add=True)
                return carry

            lax.fori_loop(0, nch, body, 0)
            plsc.subcore_barrier()

            @pl.when(cid == 0)
            def _():
                pltpu.sync_copy(acc.at[pl.ds(r0, rpw)], outA.at[pl.ds(r0, rpw)])

            @pl.when(cid == 1)
            def _():
                pltpu.sync_copy(acc.at[pl.ds(r0, rpw)], outB.at[pl.ds(r0, rpw)])

            plsc.subcore_barrier()

        one_half(xs0_hbm, out00, out01)
        one_half(xs1_hbm, out10, out11)

    return scatter_k


def _pick_kd(mpw):
    return next(c for c in (2048, 1280, 1024, 640, 512, 320, 256, 160, 128,
                            64, 32, 16, 8, 4, 2, 1) if mpw % c == 0)


def _make_degree(drows, m):
    mpw = m // NW
    kd = _pick_kd(mpw)
    nch = mpw // kd
    rpw = drows // NS
    mesh = plsc.VectorSubcoreMesh(core_axis_name="c", subcore_axis_name="s")

    @functools.partial(
        pl.kernel,
        out_type=(jax.ShapeDtypeStruct((drows, 16), jnp.float32),
                  jax.ShapeDtypeStruct((drows, 16), jnp.float32)),
        mesh=mesh,
        compiler_params=_SC_PARAMS,
        scratch_types=[
            pltpu.VMEM((kd,), jnp.int32),
            pltpu.VMEM((kd, 16), jnp.float32),
            pltpu.VMEM_SHARED((drows, 16), jnp.float32),
        ],
    )
    def degree_k(didx_hbm, ones_hbm, zer_hbm, out0, out1, idx, ones_v, dacc):
        cid = lax.axis_index("c")
        sid = lax.axis_index("s")
        wid = sid * NC + cid
        r0 = sid * rpw
        pltpu.sync_copy(ones_hbm, ones_v)
        pltpu.sync_copy(zer_hbm.at[pl.ds(r0, rpw)], dacc.at[pl.ds(r0, rpw)])
        plsc.subcore_barrier()

        def body(ch, carry):
            base = wid * mpw + ch * kd
            pltpu.sync_copy(didx_hbm.at[pl.ds(base, kd)], idx)
            pltpu.sync_copy(ones_v, dacc.at[idx], add=True)
            return carry

        lax.fori_loop(0, nch, body, 0)
        plsc.subcore_barrier()

        @pl.when(cid == 0)
        def _():
            pltpu.sync_copy(dacc.at[pl.ds(r0, rpw)], out0.at[pl.ds(r0, rpw)])

        @pl.when(cid == 1)
        def _():
            pltpu.sync_copy(dacc.at[pl.ds(r0, rpw)], out1.at[pl.ds(r0, rpw)])

    return degree_k


def _mlp_block(x, w0, b0, w1, b1):
    x = jnp.maximum(jnp.dot(x, w0, precision=_PREC) + b0, 0.0)
    return jnp.maximum(jnp.dot(x, w1, precision=_PREC) + b1, 0.0)


def _f0_body(d2, nodes, we, be, w0, b0, w1, b1, ds0, ds1,
             h_out, xs0_out, xs1_out):
    h = jnp.dot(nodes[...], we[...], precision=_PREC) + be[...]
    h_out[...] = h
    x = _mlp_block(h, w0[...], b0[...], w1[...], b1[...])
    xs = x * lax.rsqrt(ds0[...] + ds1[...] + 1.0)
    xs0_out[...] = xs[:, :d2]
    xs1_out[...] = xs[:, d2:]


def _f1_body(d2, a00, a01, a10, a11, xs0, xs1, h, dr0, dr1, lnsc, lnof,
             w0, b0, w1, b1, ds0, ds1, h_out, xs0_out, xs1_out):
    y = jnp.concatenate([a00[...] + a01[...] + xs0[...],
                         a10[...] + a11[...] + xs1[...]], axis=1)
    y = y * lax.rsqrt(dr0[...] + dr1[...] + 1.0)
    t = y + h[...]
    mean = jnp.mean(t, axis=1, keepdims=True)
    var = jnp.mean((t - mean) ** 2, axis=1, keepdims=True)
    hn = (t - mean) * lax.rsqrt(var + 1e-5) * lnsc[...] + lnof[...]
    h_out[...] = hn
    x = _mlp_block(hn, w0[...], b0[...], w1[...], b1[...])
    xs = x * lax.rsqrt(ds0[...] + ds1[...] + 1.0)
    xs0_out[...] = xs[:, :d2]
    xs1_out[...] = xs[:, d2:]


def _f2_body(n_valid, grid, a00, a01, a10, a11, xs0, xs1, h, dr0, dr1,
             lnsc, lnof, e2d, wd, bd, h_out, e_out, g_out, acc_ref):
    i = pl.program_id(0)
    y = jnp.concatenate([a00[...] + a01[...] + xs0[...],
                         a10[...] + a11[...] + xs1[...]], axis=1)
    y = y * lax.rsqrt(dr0[...] + dr1[...] + 1.0)
    t = y + h[...]
    mean = jnp.mean(t, axis=1, keepdims=True)
    var = jnp.mean((t - mean) ** 2, axis=1, keepdims=True)
    hn = (t - mean) * lax.rsqrt(var + 1e-5) * lnsc[...] + lnof[...]
    h_out[...] = hn
    e_out[...] = e2d[...] * 4.0

    blk = hn.shape[0]
    row = i * blk + lax.broadcasted_iota(jnp.int32, (blk, 1), 0)
    masked = jnp.where(row < n_valid, hn, 0.0)
    psum = jnp.sum(masked, axis=0, keepdims=True)

    @pl.when(i == 0)
    def _():
        acc_ref[...] = jnp.zeros_like(acc_ref)

    acc_ref[...] += psum

    @pl.when(i == grid - 1)
    def _():
        pooled = acc_ref[...] * (1.0 / n_valid)
        g_out[...] = jnp.dot(pooled, wd[...], precision=_PREC) + bd[...]


def kernel(nodes, edges, globals_, senders, receivers, W_embed, b_embed,
           W_mlp, b_mlp, ln_scale, ln_offset, W_dec, b_dec):
    n, d = nodes.shape
    e = senders.shape[0]
    latent = W_embed.shape[1]
    out_g = W_dec.shape[1]
    d2 = latent // 2

    n_acc = ((n + 1023) // 1024 + (0 if n % 1024 else 1)) * 1024
    if n_acc <= n:
        n_acc = n + 1024
    ndum = n_acc - n
    epad = ((e + NW * K - 1) // (NW * K)) * (NW * K)
    npad = epad - e
    drows = 2 * n_acc

    pad = (jnp.arange(npad, dtype=jnp.int32) % ndum) + n
    snd = jnp.concatenate([senders.astype(jnp.int32), pad])
    rcv = jnp.concatenate([receivers.astype(jnp.int32), pad])
    didx = jnp.concatenate([snd, rcv + n_acc])

    zer_half = jnp.zeros((n_acc, d2), jnp.float32)
    zer_deg = jnp.zeros((drows, 16), jnp.float32)
    ones16 = jnp.ones((_pick_kd(2 * epad // NW), 16), jnp.float32)

    deg_k = _make_degree(drows, 2 * epad)
    d0, d1 = deg_k(didx, ones16, zer_deg)
    ds0 = d0[:n_acc, 0:1]
    ds1 = d1[:n_acc, 0:1]
    dr0 = d0[n_acc:, 0:1]
    dr1 = d1[n_acc:, 0:1]

    nodes_pad = jnp.pad(nodes, ((0, n_acc - n), (0, 0)))
    be = b_embed.reshape(1, latent)
    bd = b_dec.reshape(1, out_g)

    grid = 8
    blk = n_acc // grid
    row_spec = pl.BlockSpec((blk, latent), lambda i: (i, 0))
    half_spec = pl.BlockSpec((blk, d2), lambda i: (i, 0))
    col_spec = pl.BlockSpec((blk, 1), lambda i: (i, 0))
    full_spec = pl.BlockSpec((d, latent), lambda i: (0, 0))
    vec_spec = pl.BlockSpec((1, latent), lambda i: (0, 0))

    half_shape = jax.ShapeDtypeStruct((n_acc, d2), jnp.float32)
    full_shape = jax.ShapeDtypeStruct((n_acc, latent), jnp.float32)

    f0 = pl.pallas_call(
        functools.partial(_f0_body, d2),
        grid=(grid,),
        in_specs=[row_spec, full_spec, vec_spec, full_spec, vec_spec,
                  full_spec, vec_spec, col_spec, col_spec],
        out_specs=[row_spec, half_spec, half_spec],
        out_shape=[full_shape, half_shape, half_shape],
    )
    h0, xs00, xs01 = f0(nodes_pad, W_embed, be,
                        W_mlp[0, 0], b_mlp[0, 0].reshape(1, -1),
                        W_mlp[0, 1], b_mlp[0, 1].reshape(1, -1), ds0, ds1)

    scat_k = _make_scatter(n_acc, epad, d2)

    a000, a001, a010, a011 = scat_k(xs00, xs01, snd, rcv, zer_half)
    f1 = pl.pallas_call(
        functools.partial(_f1_body, d2),
        grid=(grid,),
        in_specs=[half_spec, half_spec, half_spec, half_spec, half_spec,
                  half_spec, row_spec, col_spec, col_spec,
                  vec_spec, vec_spec, full_spec, vec_spec, full_spec,
                  vec_spec, col_spec, col_spec],
        out_specs=[row_spec, half_spec, half_spec],
        out_shape=[full_shape, half_shape, half_shape],
    )
    h1, xs10, xs11 = f1(a000, a001, a010, a011, xs00, xs01, h0, dr0, dr1,
                        ln_scale[0].reshape(1, -1), ln_offset[0].reshape(1, -1),
                        W_mlp[1, 0], b_mlp[1, 0].reshape(1, -1),
                        W_mlp[1, 1], b_mlp[1, 1].reshape(1, -1), ds0, ds1)

    a100, a101, a110, a111 = scat_k(xs10, xs11, snd, rcv, zer_half)

    e4r = e * edges.shape[1] // latent
    g2 = 5
    eblk = e4r // g2
    blk2 = n_acc // g2
    row2 = pl.BlockSpec((blk2, latent), lambda i: (i, 0))
    half2 = pl.BlockSpec((blk2, d2), lambda i: (i, 0))
    col2 = pl.BlockSpec((blk2, 1), lambda i: (i, 0))
    vec2 = pl.BlockSpec((1, latent), lambda i: (0, 0))
    e2d = edges.reshape(e4r, latent)

    f2 = pl.pallas_call(
        functools.partial(_f2_body, float(n), g2),
        grid=(g2,),
        in_specs=[half2, half2, half2, half2, half2, half2, row2,
                  col2, col2, vec2, vec2,
                  pl.BlockSpec((eblk, latent), lambda i: (i, 0)),
                  pl.BlockSpec((latent, out_g), lambda i: (0, 0)),
                  pl.BlockSpec((1, out_g), lambda i: (0, 0))],
        out_specs=[row2,
                   pl.BlockSpec((eblk, latent), lambda i: (i, 0)),
                   pl.BlockSpec((1, out_g), lambda i: (0, 0))],
        out_shape=[jax.ShapeDtypeStruct((n_acc, latent), jnp.float32),
                   jax.ShapeDtypeStruct((e4r, latent), jnp.float32),
                   jax.ShapeDtypeStruct((1, out_g), jnp.float32)],
        scratch_shapes=[pltpu.VMEM((1, latent), jnp.float32)],
    )
    h2, eout, g = f2(a100, a101, a110, a111, xs10, xs11, h1, dr0, dr1,
                     ln_scale[1].reshape(1, -1), ln_offset[1].reshape(1, -1),
                     e2d, W_dec, bd)

    return (h2[:n], eout.reshape(e, edges.shape[1]), g)
